# baseline (speedup 1.0000x reference)
import functools

import jax
import jax.numpy as jnp
from jax.experimental import pallas as pl
from jax.experimental.pallas import tpu as pltpu

_NUM_CODES = 8192
_EMBED_DIM = 32
_BETA = 0.25
_TILE = 256


def _vq_body(z_ref, cbT_ref, w_ref, zq_ref, idx_ref, loss_ref, effT_ref, c2_ref,
             *, n_steps, inv_nd):
    i = pl.program_id(0)
    hi = jax.lax.Precision.HIGHEST

    @pl.when(i == 0)
    def _init():
        effT = jax.lax.dot_general(
            w_ref[...], cbT_ref[...], (((1,), (0,)), ((), ())),
            preferred_element_type=jnp.float32,
            precision=jax.lax.Precision.DEFAULT)
        effT_ref[...] = effT
        c2_ref[...] = jnp.sum(effT * effT, axis=0, keepdims=True)
        loss_ref[...] = jnp.zeros((1, 1), jnp.float32)

    z = z_ref[...]
    effT = effT_ref[...]
    z2 = jnp.sum(z * z, axis=1, keepdims=True)
    zc = jax.lax.dot_general(
        z, effT, (((1,), (0,)), ((), ())),
        preferred_element_type=jnp.float32,
        precision=jax.lax.Precision.DEFAULT)
    d2 = jnp.maximum(z2 - 2.0 * zc + c2_ref[...], 0.0)
    idx = jnp.argmin(d2, axis=1).astype(jnp.int32)
    idx_ref[0, 0, :] = idx
    onehot = (jax.lax.broadcasted_iota(jnp.int32, d2.shape, 1)
              == idx[:, None]).astype(jnp.float32)
    zq = jax.lax.dot_general(
        onehot, effT, (((1,), (1,)), ((), ())),
        preferred_element_type=jnp.float32, precision=hi)
    zq_ref[...] = zq
    diff = z - zq
    loss_ref[...] += jnp.sum(diff * diff).reshape(1, 1)

    @pl.when(i == n_steps - 1)
    def _finish():
        loss_ref[...] = loss_ref[...] * ((1.0 + _BETA) * inv_nd)


def kernel(z_e, codebook, W):
    B, T, D = z_e.shape
    N = B * T
    K = codebook.shape[0]
    n_steps = N // _TILE
    z_flat = z_e.reshape(N, D)
    cbT = codebook.T

    body = functools.partial(_vq_body, n_steps=n_steps, inv_nd=1.0 / (N * D))
    zq_flat, idx3, loss = pl.pallas_call(
        body,
        grid=(n_steps,),
        in_specs=[
            pl.BlockSpec((_TILE, D), lambda i: (i, 0)),
            pl.BlockSpec((D, K), lambda i: (0, 0)),
            pl.BlockSpec((D, D), lambda i: (0, 0)),
        ],
        out_specs=[
            pl.BlockSpec((_TILE, D), lambda i: (i, 0)),
            pl.BlockSpec((1, 1, _TILE), lambda i: (i, 0, 0)),
            pl.BlockSpec((1, 1), lambda i: (0, 0)),
        ],
        out_shape=[
            jax.ShapeDtypeStruct((N, D), jnp.float32),
            jax.ShapeDtypeStruct((n_steps, 1, _TILE), jnp.int32),
            jax.ShapeDtypeStruct((1, 1), jnp.float32),
        ],
        scratch_shapes=[
            pltpu.VMEM((D, K), jnp.float32),
            pltpu.VMEM((1, K), jnp.float32),
        ],
    )(z_flat, cbT, W)

    z_q = zq_flat.reshape(B, T, D)
    total_loss = loss[0, 0]
    code_indices = idx3.reshape(B, T)
    return z_q, total_loss, code_indices


# speedup vs baseline: 2.5343x; 2.5343x over previous
import functools

import jax
import jax.numpy as jnp
from jax import lax
from jax.experimental import pallas as pl
from jax.experimental.pallas import tpu as pltpu
from jax.experimental.pallas import tpu_sc as plsc

_BETA = 0.25
_TILE = 256


def _vq_body(z_ref, cbT_ref, w_ref, idx_ref, loss_ref, eff_ref, effT_ref,
             c2_ref, *, n_steps, inv_nd):
    i = pl.program_id(0)

    @pl.when(i == 0)
    def _init():
        effT = jax.lax.dot_general(
            w_ref[...], cbT_ref[...], (((1,), (0,)), ((), ())),
            preferred_element_type=jnp.float32,
            precision=jax.lax.Precision.DEFAULT)
        effT_ref[...] = effT
        c2_ref[...] = jnp.sum(effT * effT, axis=0, keepdims=True)
        eff_ref[:, 0:effT.shape[0]] = jnp.swapaxes(effT, 0, 1)
        loss_ref[...] = jnp.zeros((1, 1), jnp.float32)

    z = z_ref[...]
    effT = effT_ref[...]
    z2 = jnp.sum(z * z, axis=1, keepdims=True)
    zc = jax.lax.dot_general(
        z, effT, (((1,), (0,)), ((), ())),
        preferred_element_type=jnp.float32,
        precision=jax.lax.Precision.DEFAULT)
    d2 = jnp.maximum(z2 - 2.0 * zc + c2_ref[...], 0.0)
    idx_ref[0, 0, :] = jnp.argmin(d2, axis=1).astype(jnp.int32)
    m = jnp.min(d2, axis=1)
    loss_ref[...] += jnp.sum(m).reshape(1, 1)

    @pl.when(i == n_steps - 1)
    def _finish():
        loss_ref[...] = loss_ref[...] * ((1.0 + _BETA) * inv_nd)


def _distance_argmin(z_flat, cbT, W):
    N, D = z_flat.shape
    K = cbT.shape[1]
    n_steps = N // _TILE
    body = functools.partial(_vq_body, n_steps=n_steps, inv_nd=1.0 / (N * D))
    return pl.pallas_call(
        body,
        grid=(n_steps,),
        in_specs=[
            pl.BlockSpec((_TILE, D), lambda i: (i, 0)),
            pl.BlockSpec((D, K), lambda i: (0, 0)),
            pl.BlockSpec((D, D), lambda i: (0, 0)),
        ],
        out_specs=[
            pl.BlockSpec((1, 1, _TILE), lambda i: (i, 0, 0)),
            pl.BlockSpec((1, 1), lambda i: (0, 0)),
            pl.BlockSpec((K, 128), lambda i: (0, 0)),
        ],
        out_shape=[
            jax.ShapeDtypeStruct((n_steps, 1, _TILE), jnp.int32),
            jax.ShapeDtypeStruct((1, 1), jnp.float32),
            jax.ShapeDtypeStruct((K, 128), jnp.float32),
        ],
        scratch_shapes=[
            pltpu.VMEM((D, K), jnp.float32),
            pltpu.VMEM((1, K), jnp.float32),
        ],
    )(z_flat, cbT, W)


def _sc_gather(eff, idx_flat):
    K, D = eff.shape
    N = idx_flat.shape[0]
    info = plsc.get_sparse_core_info()
    nw = info.num_cores * info.num_subcores
    per_w = N // nw
    mesh = plsc.VectorSubcoreMesh(core_axis_name="c", subcore_axis_name="s")

    @functools.partial(
        pl.kernel, mesh=mesh,
        out_type=jax.ShapeDtypeStruct((N, D), jnp.float32),
        scratch_types=[
            pltpu.VMEM((per_w,), jnp.int32),
            pltpu.VMEM((per_w, D), jnp.float32),
            pltpu.SemaphoreType.DMA,
        ],
    )
    def gather(table_hbm, idx_hbm, out_hbm, idx_v, rows_v, sem):
        wid = lax.axis_index("s") * info.num_cores + lax.axis_index("c")
        base = wid * per_w
        pltpu.sync_copy(idx_hbm.at[pl.ds(base, per_w)], idx_v)
        pltpu.async_copy(table_hbm.at[idx_v], rows_v, sem).wait()
        pltpu.sync_copy(rows_v, out_hbm.at[pl.ds(base, per_w)])

    return gather(eff, idx_flat)


def kernel(z_e, codebook, W):
    B, T, D = z_e.shape
    N = B * T
    z_flat = z_e.reshape(N, D)
    idx3, loss, eff_pad = _distance_argmin(z_flat, codebook.T, W)
    idx_flat = idx3.reshape(N)
    zq_pad = _sc_gather(eff_pad, idx_flat)
    return zq_pad[:, :D].reshape(B, T, D), loss[0, 0], idx3.reshape(B, T)


# speedup vs baseline: 3.6675x; 1.4471x over previous
import functools

import jax
import jax.numpy as jnp
from jax import lax
from jax.experimental import pallas as pl
from jax.experimental.pallas import tpu as pltpu
from jax.experimental.pallas import tpu_sc as plsc

_BETA = 0.25
_TILE = 256


def _split3(x):
    hi = x.astype(jnp.bfloat16).astype(jnp.float32)
    r = x - hi
    mid = r.astype(jnp.bfloat16).astype(jnp.float32)
    return hi, mid, r - mid


def _vq_body(z_ref, cbT_ref, w_ref, idx_ref, loss_ref, eff_ref, a_ref,
             *, n_steps, inv_nd):
    i = pl.program_id(0)

    @pl.when(i == 0)
    def _init():
        effT = jax.lax.dot_general(
            w_ref[...], cbT_ref[...], (((1,), (0,)), ((), ())),
            preferred_element_type=jnp.float32,
            precision=jax.lax.Precision.DEFAULT)
        k = effT.shape[1]
        c2 = jnp.sum(effT * effT, axis=0, keepdims=True)
        c2h, c2m, c2l = _split3(c2)
        a_ref[...] = jnp.concatenate(
            [-2.0 * effT, jnp.ones((3, k), jnp.float32), c2h, c2m, c2l],
            axis=0)
        eff_ref[:, 0:effT.shape[0]] = jnp.swapaxes(effT, 0, 1)
        loss_ref[...] = jnp.zeros((1, 1), jnp.float32)

    z = z_ref[...]
    z2 = jnp.sum(z * z, axis=1, keepdims=True)
    z2h, z2m, z2l = _split3(z2)
    ones = jnp.ones((z.shape[0], 3), jnp.float32)
    z_aug = jnp.concatenate([z, z2h, z2m, z2l, ones], axis=1)
    d2 = jax.lax.dot_general(
        z_aug, a_ref[...], (((1,), (0,)), ((), ())),
        preferred_element_type=jnp.float32,
        precision=jax.lax.Precision.DEFAULT)
    idx_ref[0, 0, :] = jnp.argmin(d2, axis=1).astype(jnp.int32)
    m = jnp.maximum(jnp.min(d2, axis=1), 0.0)
    loss_ref[...] += jnp.sum(m).reshape(1, 1)

    @pl.when(i == n_steps - 1)
    def _finish():
        loss_ref[...] = loss_ref[...] * ((1.0 + _BETA) * inv_nd)


def _distance_argmin(z_flat, cbT, W):
    N, D = z_flat.shape
    K = cbT.shape[1]
    n_steps = N // _TILE
    body = functools.partial(_vq_body, n_steps=n_steps, inv_nd=1.0 / (N * D))
    return pl.pallas_call(
        body,
        grid=(n_steps,),
        in_specs=[
            pl.BlockSpec((_TILE, D), lambda i: (i, 0)),
            pl.BlockSpec((D, K), lambda i: (0, 0)),
            pl.BlockSpec((D, D), lambda i: (0, 0)),
        ],
        out_specs=[
            pl.BlockSpec((1, 1, _TILE), lambda i: (i, 0, 0)),
            pl.BlockSpec((1, 1), lambda i: (0, 0)),
            pl.BlockSpec((K, 128), lambda i: (0, 0)),
        ],
        out_shape=[
            jax.ShapeDtypeStruct((n_steps, 1, _TILE), jnp.int32),
            jax.ShapeDtypeStruct((1, 1), jnp.float32),
            jax.ShapeDtypeStruct((K, 128), jnp.float32),
        ],
        scratch_shapes=[
            pltpu.VMEM((D + 6, K), jnp.float32),
        ],
    )(z_flat, cbT, W)


def _sc_gather(eff, idx_flat):
    K, D = eff.shape
    N = idx_flat.shape[0]
    info = plsc.get_sparse_core_info()
    nw = info.num_cores * info.num_subcores
    per_w = N // nw
    mesh = plsc.VectorSubcoreMesh(core_axis_name="c", subcore_axis_name="s")

    @functools.partial(
        pl.kernel, mesh=mesh,
        out_type=jax.ShapeDtypeStruct((N, D), jnp.float32),
        scratch_types=[
            pltpu.VMEM((per_w,), jnp.int32),
            pltpu.VMEM((per_w, D), jnp.float32),
            pltpu.SemaphoreType.DMA,
        ],
    )
    def gather(table_hbm, idx_hbm, out_hbm, idx_v, rows_v, sem):
        wid = lax.axis_index("s") * info.num_cores + lax.axis_index("c")
        base = wid * per_w
        pltpu.sync_copy(idx_hbm.at[pl.ds(base, per_w)], idx_v)
        pltpu.async_copy(table_hbm.at[idx_v], rows_v, sem).wait()
        pltpu.sync_copy(rows_v, out_hbm.at[pl.ds(base, per_w)])

    return gather(eff, idx_flat)


def kernel(z_e, codebook, W):
    B, T, D = z_e.shape
    N = B * T
    z_flat = z_e.reshape(N, D)
    idx3, loss, eff_pad = _distance_argmin(z_flat, codebook.T, W)
    idx_flat = idx3.reshape(N)
    zq_pad = _sc_gather(eff_pad, idx_flat)
    return zq_pad[:, :D].reshape(B, T, D), loss[0, 0], idx3.reshape(B, T)
